# Initial kernel scaffold; baseline (speedup 1.0000x reference)
import jax
import jax.numpy as jnp
from jax.experimental import pallas as pl


def kernel(hidden_states, router_weight, gate_up_proj, down_proj, shared_gate_w, shared_up_w, shared_down_w, shared_expert_gate_w):
    raise NotImplementedError("write your pallas kernel here")


# speedup vs baseline: 1.8852x; 1.8852x over previous
import functools

import jax
import jax.numpy as jnp
from jax.experimental import pallas as pl
from jax.experimental.pallas import tpu as pltpu

B, S, H = 1, 2048, 768
E, K, I, SI = 8, 2, 256, 512
T = B * S
BLK_T = 256


def _moe_block_kernel(x_ref, rw_ref, gup_ref, down_ref, sg_ref, su_ref, sd_ref,
                      seg_ref, out_ref, logits_ref):
    x = x_ref[...]

    logits = jax.lax.dot_general(x, rw_ref[...],
                                 (((1,), (1,)), ((), ())),
                                 preferred_element_type=jnp.float32)
    logits_ref[...] = logits

    lane = jax.lax.broadcasted_iota(jnp.int32, (BLK_T, E), 1)
    m1 = jnp.max(logits, axis=-1, keepdims=True)
    i1 = jnp.min(jnp.where(logits == m1, lane, E), axis=-1, keepdims=True)
    oh1 = (lane == i1)
    l2 = jnp.where(oh1, -jnp.inf, logits)
    m2 = jnp.max(l2, axis=-1, keepdims=True)
    i2 = jnp.min(jnp.where(l2 == m2, lane, E), axis=-1, keepdims=True)
    oh2 = (lane == i2)
    w1 = jax.nn.sigmoid(m1 - m2)
    w2 = 1.0 - w1
    comb = jnp.where(oh1, w1, 0.0) + jnp.where(oh2, w2, 0.0)

    acc = jnp.zeros((BLK_T, H), dtype=jnp.float32)
    for e in range(E):
        gu = jax.lax.dot_general(x, gup_ref[e],
                                 (((1,), (1,)), ((), ())),
                                 preferred_element_type=jnp.float32)
        gate = gu[:, :I]
        up = gu[:, I:]
        hmid = jax.nn.silu(gate) * up
        y = jax.lax.dot_general(hmid, down_ref[e],
                                (((1,), (1,)), ((), ())),
                                preferred_element_type=jnp.float32)
        acc = acc + comb[:, e:e + 1] * y

    sgate = jax.lax.dot_general(x, sg_ref[...], (((1,), (1,)), ((), ())),
                                preferred_element_type=jnp.float32)
    sup = jax.lax.dot_general(x, su_ref[...], (((1,), (1,)), ((), ())),
                              preferred_element_type=jnp.float32)
    smid = jax.nn.silu(sgate) * sup
    shared = jax.lax.dot_general(smid, sd_ref[...], (((1,), (1,)), ((), ())),
                                 preferred_element_type=jnp.float32)
    g = jax.nn.sigmoid(jax.lax.dot_general(x, seg_ref[...],
                                           (((1,), (1,)), ((), ())),
                                           preferred_element_type=jnp.float32))
    out_ref[...] = acc + g * shared


@jax.jit
def _run(x2d, router_weight, gate_up_proj, down_proj,
         shared_gate_w, shared_up_w, shared_down_w, shared_expert_gate_w):
    n_blk = T // BLK_T
    full = lambda shape: pl.BlockSpec(shape, lambda i: (0,) * len(shape))
    out, logits = pl.pallas_call(
        _moe_block_kernel,
        grid=(n_blk,),
        in_specs=[
            pl.BlockSpec((BLK_T, H), lambda i: (i, 0)),
            full((E, H)),
            full((E, 2 * I, H)),
            full((E, H, I)),
            full((SI, H)),
            full((SI, H)),
            full((H, SI)),
            full((1, H)),
        ],
        out_specs=[
            pl.BlockSpec((BLK_T, H), lambda i: (i, 0)),
            pl.BlockSpec((BLK_T, E), lambda i: (i, 0)),
        ],
        out_shape=[
            jax.ShapeDtypeStruct((T, H), jnp.float32),
            jax.ShapeDtypeStruct((T, E), jnp.float32),
        ],
    )(x2d, router_weight, gate_up_proj, down_proj,
      shared_gate_w, shared_up_w, shared_down_w, shared_expert_gate_w)
    return out, logits


def kernel(hidden_states, router_weight, gate_up_proj, down_proj,
           shared_gate_w, shared_up_w, shared_down_w, shared_expert_gate_w):
    b, s, h = hidden_states.shape
    x2d = hidden_states.reshape(-1, h)
    out, logits = _run(x2d, router_weight, gate_up_proj, down_proj,
                       shared_gate_w, shared_up_w, shared_down_w,
                       shared_expert_gate_w)
    return out.reshape(b, s, h), logits
